# Initial kernel scaffold; baseline (speedup 1.0000x reference)
import jax
import jax.numpy as jnp
from jax.experimental import pallas as pl


def kernel(x_num, x_cat, emb_table, weight, bias):
    raise NotImplementedError("write your pallas kernel here")


# speedup vs baseline: 2.4994x; 2.4994x over previous
import functools

import jax
import jax.numpy as jnp
from jax import lax
from jax.experimental import pallas as pl
from jax.experimental.pallas import tpu as pltpu
from jax.experimental.pallas import tpu_sc as plsc

B = 4096
DIM_NUM = 26
DIM_EMB = 128
MAX_LEN = 50
DIM_BIAS = DIM_NUM + MAX_LEN
N_DENSE = DIM_NUM + 1
N_OUT = N_DENSE + MAX_LEN
LANES = 16
NCHUNK = DIM_EMB // LANES

_NC = 2
_NS = 16
NW = _NC * _NS
BPW = B // NW


def _body(xn_hbm, xc_hbm, tab_hbm, w_hbm, b_hbm, out_hbm,
          xn_v, xc_v, w_v, b_v, row_v, sem):
    cid = lax.axis_index("c")
    sid = lax.axis_index("s")
    wid = sid * _NC + cid
    base = wid * BPW

    pltpu.sync_copy(xn_hbm.at[pl.ds(base, BPW)], xn_v)
    pltpu.sync_copy(xc_hbm.at[pl.ds(base, BPW)], xc_v)
    pltpu.sync_copy(w_hbm, w_v)
    pltpu.sync_copy(b_hbm, b_v)

    def per_row(i, carry):
        pltpu.async_copy(
            tab_hbm.at[xc_v.at[i]], row_v.at[pl.ds(N_DENSE, MAX_LEN)], sem
        ).wait()

        for c in range(NCHUNK):
            sl = pl.ds(c * LANES, LANES)
            row_v[0, sl] = w_v[0, sl]

        xs0 = xn_v[i, pl.ds(0, LANES)]
        xs1 = xn_v[i, pl.ds(LANES, LANES)]
        for j in range(1, N_DENSE):
            col = j - 1
            xs = xs0[col] if col < LANES else xs1[col - LANES]
            for c in range(NCHUNK):
                sl = pl.ds(c * LANES, LANES)
                row_v[j, sl] = w_v[j, sl] * xs + b_v[col, sl]

        def emb(l, _):
            for c in range(NCHUNK):
                sl = pl.ds(c * LANES, LANES)
                row_v[N_DENSE + l, sl] = row_v[N_DENSE + l, sl] + b_v[DIM_NUM + l, sl]
            return 0

        lax.fori_loop(0, MAX_LEN, emb, 0)

        pltpu.sync_copy(row_v, out_hbm.at[base + i])
        return 0

    lax.fori_loop(0, BPW, per_row, 0)


@jax.jit
def kernel(x_num, x_cat, emb_table, weight, bias):
    run = functools.partial(
        pl.kernel,
        mesh=plsc.VectorSubcoreMesh(core_axis_name="c", subcore_axis_name="s"),
        out_type=jax.ShapeDtypeStruct((B, N_OUT, DIM_EMB), jnp.float32),
        scratch_types=[
            pltpu.VMEM((BPW, 2 * LANES), jnp.float32),
            pltpu.VMEM((BPW, MAX_LEN), jnp.int32),
            pltpu.VMEM((N_DENSE, DIM_EMB), jnp.float32),
            pltpu.VMEM((DIM_BIAS, DIM_EMB), jnp.float32),
            pltpu.VMEM((N_OUT, DIM_EMB), jnp.float32),
            pltpu.SemaphoreType.DMA,
        ],
    )(_body)
    x_num_p = jnp.pad(x_num, ((0, 0), (0, 2 * LANES - DIM_NUM)))
    return run(x_num_p, x_cat, emb_table, weight, bias)
